# baseline (speedup 1.0000x reference)
import functools
import jax
import jax.numpy as jnp
from jax import lax
from jax.experimental import pallas as pl
from jax.experimental.pallas import tpu as pltpu, tpu_sc as plsc

_NBUF = 4
_RG = 8
_CB = 2688


def _sc_copy_add_t(d, n):
    info = plsc.get_sparse_core_info()
    NC, NS = info.num_cores, info.num_subcores
    NW = NC * NS
    nrg = d // _RG
    ncg = NW // nrg
    cols_main = (n // 128 // ncg) * 128 * ncg
    cols_g = cols_main // ncg
    edge = n - cols_main
    nch, chrem = divmod(cols_g, _CB)
    ngroups, grem = divmod(nch, _NBUF)
    assert chrem == 0 and ngroups >= 2
    mesh = plsc.VectorSubcoreMesh(core_axis_name="c", subcore_axis_name="s")

    @functools.partial(
        pl.kernel,
        mesh=mesh,
        out_type=jax.ShapeDtypeStruct((d, n), jnp.float32),
        scratch_types=(
            [pltpu.VMEM((_RG, _CB), jnp.float32) for _ in range(_NBUF)]
            + [pltpu.VMEM((_RG, 16), jnp.float32)]
            + [pltpu.SemaphoreType.DMA for _ in range(2 * _NBUF)]
        ),
    )
    def k(x_hbm, vp_hbm, out_hbm, *refs):
        bufs = refs[:_NBUF]
        val_v = refs[_NBUF]
        sin = refs[_NBUF + 1 : _NBUF + 1 + _NBUF]
        sout = refs[_NBUF + 1 + _NBUF :]

        wid = lax.axis_index("s") * NC + lax.axis_index("c")
        rg = wid // ncg
        cg = lax.rem(wid, ncg)
        r0 = rg * _RG
        c0 = cg * cols_g

        def in_cp(c, b):
            return pltpu.make_async_copy(
                x_hbm.at[pl.ds(r0, _RG), pl.ds(c0 + c * _CB, _CB)],
                bufs[b],
                sin[b],
            )

        def out_cp(c, b):
            return pltpu.make_async_copy(
                bufs[b],
                out_hbm.at[pl.ds(r0, _RG), pl.ds(c0 + c * _CB, _CB)],
                sout[b],
            )

        for b in range(_NBUF):
            in_cp(b, b).start()
        for b in range(_NBUF):
            in_cp(b, b).wait()
            if b == 0:
                @pl.when(cg == 0)
                def _():
                    pltpu.sync_copy(vp_hbm.at[pl.ds(r0, _RG)], val_v)
                    for r in range(_RG):
                        sl = pl.ds(0, 16)
                        bufs[0][r, sl] = bufs[0][r, sl] + val_v[r, sl]
            out_cp(b, b).start()

        @pl.loop(1, ngroups)
        def _(g):
            cbase = g * _NBUF
            for b in range(_NBUF):
                out_cp(cbase - _NBUF + b, b).wait()
                in_cp(cbase + b, b).start()
            for b in range(_NBUF):
                in_cp(cbase + b, b).wait()
                out_cp(cbase + b, b).start()

        for b in range(_NBUF):
            out_cp((ngroups - 1) * _NBUF + b, b).wait()

        for e in range(grem):
            ce = ngroups * _NBUF + e
            pltpu.sync_copy(
                x_hbm.at[pl.ds(r0, _RG), pl.ds(c0 + ce * _CB, _CB)], bufs[0]
            )
            pltpu.sync_copy(
                bufs[0], out_hbm.at[pl.ds(r0, _RG), pl.ds(c0 + ce * _CB, _CB)]
            )

    return k


def kernel(main_tensor, value):
    n, d = main_tensor.shape
    xt = main_tensor.T
    vpad = jnp.zeros((d, 16), dtype=value.dtype).at[:, : value.shape[0]].set(value.T)
    out = _sc_copy_add_t(d, n)(xt, vpad).T
    cols_main = (n // 128 // 4) * 128 * 4
    if cols_main < n:
        tail_rows = jax.lax.slice(main_tensor, (cols_main, 0), (n, d))
        out = jax.lax.dynamic_update_slice(out, tail_rows, (cols_main, 0))
    return out


# speedup vs baseline: 1.0003x; 1.0003x over previous
import functools
import jax
import jax.numpy as jnp
from jax import lax
from jax.experimental import pallas as pl
from jax.experimental.pallas import tpu as pltpu, tpu_sc as plsc

_NBUF = 3
_RG = 8
_CB = 2688


def _sc_copy_add_t(d, n):
    info = plsc.get_sparse_core_info()
    NC, NS = info.num_cores, info.num_subcores
    NW = NC * NS
    nrg = d // _RG
    ncg = NW // nrg
    cols_main = (n // 128 // ncg) * 128 * ncg
    cols_g = cols_main // ncg
    edge = n - cols_main
    nch, chrem = divmod(cols_g, _CB)
    ngroups, grem = divmod(nch, _NBUF)
    assert chrem == 0 and ngroups >= 2
    mesh = plsc.VectorSubcoreMesh(core_axis_name="c", subcore_axis_name="s")

    @functools.partial(
        pl.kernel,
        mesh=mesh,
        out_type=jax.ShapeDtypeStruct((d, n), jnp.float32),
        scratch_types=(
            [pltpu.VMEM((_RG, _CB), jnp.float32) for _ in range(_NBUF)]
            + [pltpu.VMEM((_RG, 16), jnp.float32)]
            + [pltpu.SemaphoreType.DMA for _ in range(2 * _NBUF)]
        ),
    )
    def k(x_hbm, vp_hbm, out_hbm, *refs):
        bufs = refs[:_NBUF]
        val_v = refs[_NBUF]
        sin = refs[_NBUF + 1 : _NBUF + 1 + _NBUF]
        sout = refs[_NBUF + 1 + _NBUF :]

        wid = lax.axis_index("s") * NC + lax.axis_index("c")
        rg = wid // ncg
        cg = lax.rem(wid, ncg)
        r0 = rg * _RG
        c0 = cg * cols_g

        def in_cp(c, b):
            return pltpu.make_async_copy(
                x_hbm.at[pl.ds(r0, _RG), pl.ds(c0 + c * _CB, _CB)],
                bufs[b],
                sin[b],
            )

        def out_cp(c, b):
            return pltpu.make_async_copy(
                bufs[b],
                out_hbm.at[pl.ds(r0, _RG), pl.ds(c0 + c * _CB, _CB)],
                sout[b],
            )

        for b in range(_NBUF):
            in_cp(b, b).start()
        for b in range(_NBUF):
            in_cp(b, b).wait()
            if b == 0:
                @pl.when(cg == 0)
                def _():
                    pltpu.sync_copy(vp_hbm.at[pl.ds(r0, _RG)], val_v)
                    for r in range(_RG):
                        sl = pl.ds(0, 16)
                        bufs[0][r, sl] = bufs[0][r, sl] + val_v[r, sl]
            out_cp(b, b).start()

        @pl.loop(1, ngroups)
        def _(g):
            cbase = g * _NBUF
            for b in range(_NBUF):
                out_cp(cbase - _NBUF + b, b).wait()
                in_cp(cbase + b, b).start()
            for b in range(_NBUF):
                in_cp(cbase + b, b).wait()
                out_cp(cbase + b, b).start()

        for b in range(_NBUF):
            out_cp((ngroups - 1) * _NBUF + b, b).wait()

        for e in range(grem):
            ce = ngroups * _NBUF + e
            pltpu.sync_copy(
                x_hbm.at[pl.ds(r0, _RG), pl.ds(c0 + ce * _CB, _CB)], bufs[0]
            )
            pltpu.sync_copy(
                bufs[0], out_hbm.at[pl.ds(r0, _RG), pl.ds(c0 + ce * _CB, _CB)]
            )

    return k


def kernel(main_tensor, value):
    n, d = main_tensor.shape
    xt = main_tensor.T
    vpad = jnp.zeros((d, 16), dtype=value.dtype).at[:, : value.shape[0]].set(value.T)
    out = _sc_copy_add_t(d, n)(xt, vpad).T
    cols_main = (n // 128 // 4) * 128 * 4
    if cols_main < n:
        tail_rows = jax.lax.slice(main_tensor, (cols_main, 0), (n, d))
        out = jax.lax.dynamic_update_slice(out, tail_rows, (cols_main, 0))
    return out
